# baseline (speedup 1.0000x reference)
import functools

import jax
import jax.numpy as jnp
from jax import lax
from jax.experimental import pallas as pl
from jax.experimental.pallas import tpu as pltpu
from jax.experimental.pallas import tpu_sc as plsc

NV = 5
DV = 16
TD = NV * DV
ROWW = 96
_NC = 2
_NS = 16
_L = 16
_C = 80
_BN = 1000


def _tables_body(x_ref, w0_ref, b0_ref, w1_ref, b1_ref, wt_ref, bt_ref,
                 wsh_ref, bsh_ref, t_ref, sh_ref):
    dn = (((1,), (1,)), ((), ()))
    h = jnp.maximum(
        lax.dot_general(x_ref[...], w0_ref[...], dn,
                        preferred_element_type=jnp.float32) + b0_ref[...], 0.0)
    h = jnp.maximum(
        lax.dot_general(h, w1_ref[...], dn,
                        preferred_element_type=jnp.float32) + b1_ref[...], 0.0)
    t_ref[...] = jnp.maximum(
        lax.dot_general(h, wt_ref[...], dn,
                        preferred_element_type=jnp.float32) + bt_ref[...], 0.0)
    sh_ref[...] = jnp.maximum(
        lax.dot_general(h, wsh_ref[...], dn,
                        preferred_element_type=jnp.float32) + bsh_ref[...], 0.0)


def _sc_edge_body(src_hbm, dst_hbm, t_hbm, sh_hbm, out0, out1,
                  srcv, dstv, dstm, trow, shrow, mrow, spacc,
                  sem_t, sem_s, sem_m0, sem_m1):
    cid = lax.axis_index("c")
    sid = lax.axis_index("s")
    wid = cid * _NS + sid
    n_pad = out0.shape[0]
    nrows = src_hbm.shape[0]
    kchunks = nrows // (_NC * _NS)
    rows_per_tile = n_pad // _NS
    lane = lax.iota(jnp.int32, _L)
    zero16 = jnp.zeros((_L,), jnp.float32)
    sem_m = (sem_m0, sem_m1)
    wbase = wid * kchunks

    def _zrow(i, carry):
        for b in range(2):
            for j in range(ROWW // _L):
                mrow[b, i, pl.ds(_L * j, _L)] = zero16
        return carry
    zi16 = jnp.zeros((_L,), jnp.int32)
    for b in range(2):
        for j in range(_C // _L):
            dstm[b, pl.ds(_L * j, _L)] = zi16
    lax.fori_loop(0, _C, _zrow, 0)
    base_r = sid * rows_per_tile
    nfull = rows_per_tile // _C
    rem = rows_per_tile - nfull * _C
    for k in range(nfull):
        pltpu.sync_copy(mrow.at[0], spacc.at[pl.ds(base_r + k * _C, _C)])
    if rem:
        pltpu.sync_copy(mrow.at[0, pl.ds(0, rem)],
                        spacc.at[pl.ds(base_r + nfull * _C, rem)])
    plsc.subcore_barrier()

    dn = lax.GatherDimensionNumbers(
        offset_dims=(), collapsed_slice_dims=(0,), start_index_map=(0,))

    def _start(k, b):
        pltpu.sync_copy(src_hbm.at[wbase + k], srcv.at[b])
        pltpu.sync_copy(dst_hbm.at[wbase + k], dstv.at[b])
        pltpu.async_copy(t_hbm.at[dstv.at[b]], trow.at[b], sem_t)
        pltpu.async_copy(sh_hbm.at[srcv.at[b]], shrow.at[b], sem_s)

    def _wait(b):
        pltpu.make_async_copy(t_hbm.at[dstv.at[b]], trow.at[b],
                              sem_t).wait()
        pltpu.make_async_copy(sh_hbm.at[srcv.at[b]], shrow.at[b],
                              sem_s).wait()

    def _scatter_start(b):
        pltpu.async_copy(mrow.at[b], spacc.at[dstm.at[b]], sem_m[b],
                         add=True)

    def _scatter_wait(b):
        pltpu.make_async_copy(mrow.at[b], spacc.at[dstm.at[b]],
                              sem_m[b]).wait()

    def _compute(b):
        @plsc.parallel_loop(0, _C, 1, unroll=4)
        def _edge(i):
            pacc = zero16
            for v in range(NV):
                tv = trow[b, i, pl.ds(DV * v, DV)]
                sv = shrow[b, i, pl.ds(DV * v, DV)]
                r = tv * sv
                for step in (8, 4, 2, 1):
                    idx = (lane ^ step).reshape(_L, 1)
                    r = r + lax.gather(
                        r, idx, dn, (1,),
                        mode=lax.GatherScatterMode.PROMISE_IN_BOUNDS)
                pv = jnp.exp(r)
                hv = shrow[b, i, pl.ds(TD + DV * v, DV)]
                mrow[b, i, pl.ds(DV * v, DV)] = pv * hv
                pacc = jnp.where(lane == v, pv, pacc)
            mrow[b, i, pl.ds(TD, _L)] = pacc

        for j in range(_C // _L):
            dstm[b, pl.ds(_L * j, _L)] = dstv[b, pl.ds(_L * j, _L)]

    _scatter_start(0)
    _scatter_start(1)
    _start(0, 0)

    @pl.loop(0, kchunks - 1, step=2)
    def _pair(k):
        _start(k + 1, 1)
        _wait(0)
        _scatter_wait(0)
        _compute(0)
        _scatter_start(0)
        _start(k + 2, 0)
        _wait(1)
        _scatter_wait(1)
        _compute(1)
        _scatter_start(1)

    _wait(0)
    _scatter_wait(0)
    _compute(0)
    _scatter_start(0)
    _scatter_wait(0)
    _scatter_wait(1)
    plsc.subcore_barrier()

    @pl.when(cid == 0)
    def _():
        pltpu.sync_copy(spacc.at[pl.ds(base_r, rows_per_tile)],
                        out0.at[pl.ds(base_r, rows_per_tile)])

    @pl.when(cid == 1)
    def _():
        pltpu.sync_copy(spacc.at[pl.ds(base_r, rows_per_tile)],
                        out1.at[pl.ds(base_r, rows_per_tile)])


def _finish_body(u0_ref, u1_ref, t_ref, sh_ref, wo_ref, bo_ref,
                 wout_ref, bout_ref, o_ref):
    acc = u0_ref[...] + u1_ref[...]
    tb = t_ref[...]
    shb = sh_ref[...]
    aggacc = jnp.zeros((o_ref.shape[0], DV), jnp.float32)
    for v in range(NV):
        prodv = tb[:, DV * v:DV * (v + 1)] * shb[:, DV * v:DV * (v + 1)]
        pv = jnp.exp(jnp.sum(prodv, axis=1, keepdims=True))
        hnv = shb[:, TD + DV * v:TD + DV * (v + 1)]
        uv = acc[:, DV * v:DV * (v + 1)] + pv * hnv
        normv = acc[:, TD + v:TD + v + 1] + pv + 1e-12
        aggacc = aggacc + uv / normv
    agg = aggacc * (1.0 / NV)
    dn = (((1,), (1,)), ((), ()))
    out = jnp.maximum(
        lax.dot_general(agg, wo_ref[...], dn,
                        preferred_element_type=jnp.float32) + bo_ref[...], 0.0)
    o_ref[...] = lax.dot_general(
        out, wout_ref[...], dn,
        preferred_element_type=jnp.float32) + bout_ref[...]


def kernel(x, edge_index, W_emb0, b_emb0, W_emb1, b_emb1, W_t, b_t,
           W_s, b_s, W_h, b_h, W_o, b_o, W_out, b_out):
    n, d = x.shape
    e = edge_index.shape[1]
    nblk = n // _BN

    W_sh = jnp.concatenate([W_s, W_h], axis=0)
    b_sh = jnp.concatenate([b_s, b_h], axis=0)

    full = lambda shape: pl.BlockSpec(shape, lambda i: (0, 0))
    t_tab, sh_tab = pl.pallas_call(
        _tables_body,
        grid=(nblk,),
        in_specs=[
            pl.BlockSpec((_BN, d), lambda i: (i, 0)),
            full(W_emb0.shape), full((1, 128)),
            full(W_emb1.shape), full((1, 128)),
            full(W_t.shape), full((1, TD)),
            full(W_sh.shape), full((1, 2 * TD)),
        ],
        out_specs=[
            pl.BlockSpec((_BN, TD), lambda i: (i, 0)),
            pl.BlockSpec((_BN, 2 * TD), lambda i: (i, 0)),
        ],
        out_shape=[
            jax.ShapeDtypeStruct((n, TD), jnp.float32),
            jax.ShapeDtypeStruct((n, 2 * TD), jnp.float32),
        ],
    )(x, W_emb0, b_emb0.reshape(1, -1), W_emb1, b_emb1.reshape(1, -1),
      W_t, b_t.reshape(1, -1), W_sh, b_sh.reshape(1, -1))

    n_pad = ((n + _NS * 8 - 1) // (_NS * 8)) * (_NS * 8)
    mesh = plsc.VectorSubcoreMesh(core_axis_name="c", subcore_axis_name="s")
    sc_edge = functools.partial(
        pl.kernel,
        out_type=[
            jax.ShapeDtypeStruct((n_pad, ROWW), jnp.float32),
            jax.ShapeDtypeStruct((n_pad, ROWW), jnp.float32),
        ],
        mesh=mesh,
        compiler_params=pltpu.CompilerParams(use_tc_tiling_on_sc=False),
        scratch_types=[
            pltpu.VMEM((2, _C), jnp.int32),
            pltpu.VMEM((2, _C), jnp.int32),
            pltpu.VMEM((2, _C), jnp.int32),
            pltpu.VMEM((2, _C, TD), jnp.float32),
            pltpu.VMEM((2, _C, 2 * TD), jnp.float32),
            pltpu.VMEM((2, _C, ROWW), jnp.float32),
            pltpu.VMEM_SHARED((n_pad, ROWW), jnp.float32),
            pltpu.SemaphoreType.DMA,
            pltpu.SemaphoreType.DMA,
            pltpu.SemaphoreType.DMA,
            pltpu.SemaphoreType.DMA,
        ],
    )(_sc_edge_body)
    u0, u1 = sc_edge(edge_index[0].reshape(-1, _C),
                     edge_index[1].reshape(-1, _C), t_tab, sh_tab)

    logits = pl.pallas_call(
        _finish_body,
        grid=(nblk,),
        in_specs=[
            pl.BlockSpec((_BN, ROWW), lambda i: (i, 0)),
            pl.BlockSpec((_BN, ROWW), lambda i: (i, 0)),
            pl.BlockSpec((_BN, TD), lambda i: (i, 0)),
            pl.BlockSpec((_BN, 2 * TD), lambda i: (i, 0)),
            full(W_o.shape), full((1, W_o.shape[0])),
            full(W_out.shape), full((1, W_out.shape[0])),
        ],
        out_specs=pl.BlockSpec((_BN, W_out.shape[0]), lambda i: (i, 0)),
        out_shape=jax.ShapeDtypeStruct((n, W_out.shape[0]), jnp.float32),
    )(u0, u1, t_tab, sh_tab, W_o, b_o.reshape(1, -1),
      W_out, b_out.reshape(1, -1))
    return logits


# speedup vs baseline: 1.1372x; 1.1372x over previous
import functools

import jax
import jax.numpy as jnp
from jax import lax
from jax.experimental import pallas as pl
from jax.experimental.pallas import tpu as pltpu
from jax.experimental.pallas import tpu_sc as plsc

NV = 5
DV = 16
TD = NV * DV
ROWW = 88
_NC = 2
_NS = 16
_L = 16
_C = 80
_BN = 1000


def _tables_body(x_ref, w0_ref, b0_ref, w1_ref, b1_ref, wt_ref, bt_ref,
                 wsh_ref, bsh_ref, t_ref, sh_ref):
    dn = (((1,), (1,)), ((), ()))
    h = jnp.maximum(
        lax.dot_general(x_ref[...], w0_ref[...], dn,
                        preferred_element_type=jnp.float32) + b0_ref[...], 0.0)
    h = jnp.maximum(
        lax.dot_general(h, w1_ref[...], dn,
                        preferred_element_type=jnp.float32) + b1_ref[...], 0.0)
    t_ref[...] = jnp.maximum(
        lax.dot_general(h, wt_ref[...], dn,
                        preferred_element_type=jnp.float32) + bt_ref[...], 0.0)
    sh_ref[...] = jnp.maximum(
        lax.dot_general(h, wsh_ref[...], dn,
                        preferred_element_type=jnp.float32) + bsh_ref[...], 0.0)


def _sc_edge_body(src_hbm, dst_hbm, t_hbm, sh_hbm, out0, out1,
                  srcall, dstall, trow, shrow, mrow, spacc,
                  sem_t, sem_s, sem_m0, sem_m1):
    cid = lax.axis_index("c")
    sid = lax.axis_index("s")
    wid = cid * _NS + sid
    n_pad = out0.shape[0]
    nrows = src_hbm.shape[0]
    kchunks = nrows // (_NC * _NS)
    rows_per_tile = n_pad // _NS
    lane = lax.iota(jnp.int32, _L)
    zero16 = jnp.zeros((_L,), jnp.float32)
    sem_m = (sem_m0, sem_m1)
    wbase = wid * kchunks

    def _zrow(i, carry):
        for b in range(2):
            for off in (0, 16, 32, 48, 64, ROWW - _L):
                mrow[b, i, pl.ds(off, _L)] = zero16
        return carry
    lax.fori_loop(0, _C, _zrow, 0)
    base_r = sid * rows_per_tile
    nfull = rows_per_tile // _C
    rem = rows_per_tile - nfull * _C
    for k in range(nfull):
        pltpu.sync_copy(mrow.at[0], spacc.at[pl.ds(base_r + k * _C, _C)])
    if rem:
        pltpu.sync_copy(mrow.at[0, pl.ds(0, rem)],
                        spacc.at[pl.ds(base_r + nfull * _C, rem)])
    plsc.subcore_barrier()

    pltpu.sync_copy(src_hbm.at[pl.ds(wbase, kchunks)], srcall)
    pltpu.sync_copy(dst_hbm.at[pl.ds(wbase, kchunks)], dstall)

    dn = lax.GatherDimensionNumbers(
        offset_dims=(), collapsed_slice_dims=(0,), start_index_map=(0,))

    def _start(k, b):
        pltpu.async_copy(t_hbm.at[dstall.at[k]], trow.at[b], sem_t)
        pltpu.async_copy(sh_hbm.at[srcall.at[k]], shrow.at[b], sem_s)

    def _wait(k, b):
        pltpu.make_async_copy(t_hbm.at[dstall.at[k]], trow.at[b],
                              sem_t).wait()
        pltpu.make_async_copy(sh_hbm.at[srcall.at[k]], shrow.at[b],
                              sem_s).wait()

    def _scatter_start(k, b):
        pltpu.async_copy(mrow.at[b], spacc.at[dstall.at[k]], sem_m[b],
                         add=True)

    def _scatter_wait(k, b):
        pltpu.make_async_copy(mrow.at[b], spacc.at[dstall.at[k]],
                              sem_m[b]).wait()

    def _compute(b):
        xor8 = (lane ^ 8).reshape(_L, 1)

        @plsc.parallel_loop(0, _C, 1, unroll=4)
        def _edge(i):
            pacc = zero16
            mv4 = zero16
            for v in range(NV):
                tv = trow[b, i, pl.ds(DV * v, DV)]
                sv = shrow[b, i, pl.ds(DV * v, DV)]
                r = tv * sv
                for step in (8, 4, 2, 1):
                    idx = (lane ^ step).reshape(_L, 1)
                    r = r + lax.gather(
                        r, idx, dn, (1,),
                        mode=lax.GatherScatterMode.PROMISE_IN_BOUNDS)
                pv = jnp.exp(r)
                hv = shrow[b, i, pl.ds(TD + DV * v, DV)]
                mv = pv * hv
                mrow[b, i, pl.ds(DV * v, DV)] = mv
                if v == NV - 1:
                    mv4 = mv
                pacc = jnp.where(lane == v, pv, pacc)
            m4swap = lax.gather(mv4, xor8, dn, (1,),
                                mode=lax.GatherScatterMode.PROMISE_IN_BOUNDS)
            pswap = lax.gather(pacc, xor8, dn, (1,),
                               mode=lax.GatherScatterMode.PROMISE_IN_BOUNDS)
            mrow[b, i, pl.ds(ROWW - _L, _L)] = jnp.where(
                lane < 8, m4swap, pswap)

    _scatter_start(0, 0)
    _scatter_start(0, 1)
    _start(0, 0)

    @pl.loop(0, kchunks - 1, step=2)
    def _pair(k):
        _start(k + 1, 1)
        _wait(k, 0)
        _scatter_wait(k, 0)
        _compute(0)
        _scatter_start(k, 0)
        _start(k + 2, 0)
        _wait(k + 1, 1)
        _scatter_wait(k + 1, 1)
        _compute(1)
        _scatter_start(k + 1, 1)

    _wait(kchunks - 1, 0)
    _scatter_wait(kchunks - 1, 0)
    _compute(0)
    _scatter_start(kchunks - 1, 0)
    _scatter_wait(kchunks - 1, 0)
    _scatter_wait(kchunks - 1, 1)
    plsc.subcore_barrier()

    @pl.when(cid == 0)
    def _():
        pltpu.sync_copy(spacc.at[pl.ds(base_r, rows_per_tile)],
                        out0.at[pl.ds(base_r, rows_per_tile)])

    @pl.when(cid == 1)
    def _():
        pltpu.sync_copy(spacc.at[pl.ds(base_r, rows_per_tile)],
                        out1.at[pl.ds(base_r, rows_per_tile)])


def _finish_body(u0_ref, u1_ref, t_ref, sh_ref, wo_ref, bo_ref,
                 wout_ref, bout_ref, o_ref):
    acc = u0_ref[...] + u1_ref[...]
    tb = t_ref[...]
    shb = sh_ref[...]
    aggacc = jnp.zeros((o_ref.shape[0], DV), jnp.float32)
    for v in range(NV):
        prodv = tb[:, DV * v:DV * (v + 1)] * shb[:, DV * v:DV * (v + 1)]
        pv = jnp.exp(jnp.sum(prodv, axis=1, keepdims=True))
        hnv = shb[:, TD + DV * v:TD + DV * (v + 1)]
        uv = acc[:, DV * v:DV * (v + 1)] + pv * hnv
        normv = acc[:, TD + v:TD + v + 1] + pv + 1e-12
        aggacc = aggacc + uv / normv
    agg = aggacc * (1.0 / NV)
    dn = (((1,), (1,)), ((), ()))
    out = jnp.maximum(
        lax.dot_general(agg, wo_ref[...], dn,
                        preferred_element_type=jnp.float32) + bo_ref[...], 0.0)
    o_ref[...] = lax.dot_general(
        out, wout_ref[...], dn,
        preferred_element_type=jnp.float32) + bout_ref[...]


def kernel(x, edge_index, W_emb0, b_emb0, W_emb1, b_emb1, W_t, b_t,
           W_s, b_s, W_h, b_h, W_o, b_o, W_out, b_out):
    n, d = x.shape
    e = edge_index.shape[1]
    nblk = n // _BN

    W_sh = jnp.concatenate([W_s, W_h], axis=0)
    b_sh = jnp.concatenate([b_s, b_h], axis=0)

    full = lambda shape: pl.BlockSpec(shape, lambda i: (0, 0))
    t_tab, sh_tab = pl.pallas_call(
        _tables_body,
        grid=(nblk,),
        in_specs=[
            pl.BlockSpec((_BN, d), lambda i: (i, 0)),
            full(W_emb0.shape), full((1, 128)),
            full(W_emb1.shape), full((1, 128)),
            full(W_t.shape), full((1, TD)),
            full(W_sh.shape), full((1, 2 * TD)),
        ],
        out_specs=[
            pl.BlockSpec((_BN, TD), lambda i: (i, 0)),
            pl.BlockSpec((_BN, 2 * TD), lambda i: (i, 0)),
        ],
        out_shape=[
            jax.ShapeDtypeStruct((n, TD), jnp.float32),
            jax.ShapeDtypeStruct((n, 2 * TD), jnp.float32),
        ],
    )(x, W_emb0, b_emb0.reshape(1, -1), W_emb1, b_emb1.reshape(1, -1),
      W_t, b_t.reshape(1, -1), W_sh, b_sh.reshape(1, -1))

    n_pad = ((n + _NS * 8 - 1) // (_NS * 8)) * (_NS * 8)
    mesh = plsc.VectorSubcoreMesh(core_axis_name="c", subcore_axis_name="s")
    sc_edge = functools.partial(
        pl.kernel,
        out_type=[
            jax.ShapeDtypeStruct((n_pad, ROWW), jnp.float32),
            jax.ShapeDtypeStruct((n_pad, ROWW), jnp.float32),
        ],
        mesh=mesh,
        compiler_params=pltpu.CompilerParams(use_tc_tiling_on_sc=False),
        scratch_types=[
            pltpu.VMEM((e // _C // (_NC * _NS), _C), jnp.int32),
            pltpu.VMEM((e // _C // (_NC * _NS), _C), jnp.int32),
            pltpu.VMEM((2, _C, TD), jnp.float32),
            pltpu.VMEM((2, _C, 2 * TD), jnp.float32),
            pltpu.VMEM((2, _C, ROWW), jnp.float32),
            pltpu.VMEM_SHARED((n_pad, ROWW), jnp.float32),
            pltpu.SemaphoreType.DMA,
            pltpu.SemaphoreType.DMA,
            pltpu.SemaphoreType.DMA,
            pltpu.SemaphoreType.DMA,
        ],
    )(_sc_edge_body)
    u0, u1 = sc_edge(edge_index[0].reshape(-1, _C),
                     edge_index[1].reshape(-1, _C), t_tab, sh_tab)

    logits = pl.pallas_call(
        _finish_body,
        grid=(nblk,),
        in_specs=[
            pl.BlockSpec((_BN, ROWW), lambda i: (i, 0)),
            pl.BlockSpec((_BN, ROWW), lambda i: (i, 0)),
            pl.BlockSpec((_BN, TD), lambda i: (i, 0)),
            pl.BlockSpec((_BN, 2 * TD), lambda i: (i, 0)),
            full(W_o.shape), full((1, W_o.shape[0])),
            full(W_out.shape), full((1, W_out.shape[0])),
        ],
        out_specs=pl.BlockSpec((_BN, W_out.shape[0]), lambda i: (i, 0)),
        out_shape=jax.ShapeDtypeStruct((n, W_out.shape[0]), jnp.float32),
    )(u0, u1, t_tab, sh_tab, W_o, b_o.reshape(1, -1),
      W_out, b_out.reshape(1, -1))
    return logits


# speedup vs baseline: 1.1740x; 1.0323x over previous
import functools

import jax
import jax.numpy as jnp
from jax import lax
from jax.experimental import pallas as pl
from jax.experimental.pallas import tpu as pltpu
from jax.experimental.pallas import tpu_sc as plsc

NV = 5
DV = 16
TD = NV * DV
ROWW = 96
_NC = 2
_NS = 16
_L = 16
_C = 80
_BN = 400


def _tables_body(x_ref, w0_ref, b0_ref, w1_ref, b1_ref, wt_ref, bt_ref,
                 wsh_ref, bsh_ref, t_ref, sh_ref):
    dn = (((1,), (1,)), ((), ()))
    h = jnp.maximum(
        lax.dot_general(x_ref[...], w0_ref[...], dn,
                        preferred_element_type=jnp.float32) + b0_ref[...], 0.0)
    h = jnp.maximum(
        lax.dot_general(h, w1_ref[...], dn,
                        preferred_element_type=jnp.float32) + b1_ref[...], 0.0)
    t_ref[...] = jnp.maximum(
        lax.dot_general(h, wt_ref[...], dn,
                        preferred_element_type=jnp.float32) + bt_ref[...], 0.0)
    sh_ref[...] = jnp.maximum(
        lax.dot_general(h, wsh_ref[...], dn,
                        preferred_element_type=jnp.float32) + bsh_ref[...], 0.0)


def _sc_edge_body(src_hbm, dst_hbm, t_hbm, sh_hbm, out0, out1,
                  srcall, dstall, trow, shrow, mrow, spacc, sem_t, sem_s):
    cid = lax.axis_index("c")
    sid = lax.axis_index("s")
    wid = cid * _NS + sid
    n_pad = out0.shape[0]
    nrows = src_hbm.shape[0]
    kchunks = nrows // (_NC * _NS)
    rows_per_tile = n_pad // _NS
    lane = lax.iota(jnp.int32, _L)
    zero16 = jnp.zeros((_L,), jnp.float32)
    wbase = wid * kchunks

    def _zrow(i, carry):
        for j in range(ROWW // _L):
            mrow[i, pl.ds(_L * j, _L)] = zero16
        return carry
    lax.fori_loop(0, _C, _zrow, 0)
    base_r = sid * rows_per_tile
    nfull = rows_per_tile // _C
    rem = rows_per_tile - nfull * _C
    for k in range(nfull):
        pltpu.sync_copy(mrow, spacc.at[pl.ds(base_r + k * _C, _C)])
    if rem:
        pltpu.sync_copy(mrow.at[pl.ds(0, rem)],
                        spacc.at[pl.ds(base_r + nfull * _C, rem)])
    plsc.subcore_barrier()

    pltpu.sync_copy(src_hbm.at[pl.ds(wbase, kchunks)], srcall)
    pltpu.sync_copy(dst_hbm.at[pl.ds(wbase, kchunks)], dstall)

    dn = lax.GatherDimensionNumbers(
        offset_dims=(), collapsed_slice_dims=(0,), start_index_map=(0,))

    def _start(k, b):
        pltpu.async_copy(t_hbm.at[dstall.at[k]], trow.at[b], sem_t)
        pltpu.async_copy(sh_hbm.at[srcall.at[k]], shrow.at[b], sem_s)

    def _wait(k, b):
        pltpu.make_async_copy(t_hbm.at[dstall.at[k]], trow.at[b],
                              sem_t).wait()
        pltpu.make_async_copy(sh_hbm.at[srcall.at[k]], shrow.at[b],
                              sem_s).wait()

    def _compute_scatter(k, b):
        @plsc.parallel_loop(0, _C, 1, unroll=4)
        def _edge(i):
            pacc = zero16
            for v in range(NV):
                tv = trow[b, i, pl.ds(DV * v, DV)]
                sv = shrow[b, i, pl.ds(DV * v, DV)]
                r = tv * sv
                for step in (8, 4, 2, 1):
                    idx = (lane ^ step).reshape(_L, 1)
                    r = r + lax.gather(
                        r, idx, dn, (1,),
                        mode=lax.GatherScatterMode.PROMISE_IN_BOUNDS)
                pv = jnp.exp(r)
                hv = shrow[b, i, pl.ds(TD + DV * v, DV)]
                mrow[i, pl.ds(DV * v, DV)] = pv * hv
                pacc = jnp.where(lane == v, pv, pacc)
            mrow[i, pl.ds(TD, _L)] = pacc

        pltpu.sync_copy(mrow, spacc.at[dstall.at[k]], add=True)

    _start(0, 0)

    @pl.loop(0, kchunks - 1, step=2)
    def _pair(k):
        _start(k + 1, 1)
        _wait(k, 0)
        _compute_scatter(k, 0)
        _start(k + 2, 0)
        _wait(k + 1, 1)
        _compute_scatter(k + 1, 1)

    _wait(kchunks - 1, 0)
    _compute_scatter(kchunks - 1, 0)
    plsc.subcore_barrier()

    @pl.when(cid == 0)
    def _():
        pltpu.sync_copy(spacc.at[pl.ds(base_r, rows_per_tile)],
                        out0.at[pl.ds(base_r, rows_per_tile)])

    @pl.when(cid == 1)
    def _():
        pltpu.sync_copy(spacc.at[pl.ds(base_r, rows_per_tile)],
                        out1.at[pl.ds(base_r, rows_per_tile)])


def _finish_body(u0_ref, u1_ref, t_ref, sh_ref, wo_ref, bo_ref,
                 wout_ref, bout_ref, o_ref):
    acc = u0_ref[...] + u1_ref[...]
    tb = t_ref[...]
    shb = sh_ref[...]
    aggacc = jnp.zeros((o_ref.shape[0], DV), jnp.float32)
    for v in range(NV):
        prodv = tb[:, DV * v:DV * (v + 1)] * shb[:, DV * v:DV * (v + 1)]
        pv = jnp.exp(jnp.sum(prodv, axis=1, keepdims=True))
        hnv = shb[:, TD + DV * v:TD + DV * (v + 1)]
        uv = acc[:, DV * v:DV * (v + 1)] + pv * hnv
        normv = acc[:, TD + v:TD + v + 1] + pv + 1e-12
        aggacc = aggacc + uv / normv
    agg = aggacc * (1.0 / NV)
    dn = (((1,), (1,)), ((), ()))
    out = jnp.maximum(
        lax.dot_general(agg, wo_ref[...], dn,
                        preferred_element_type=jnp.float32) + bo_ref[...], 0.0)
    o_ref[...] = lax.dot_general(
        out, wout_ref[...], dn,
        preferred_element_type=jnp.float32) + bout_ref[...]


def kernel(x, edge_index, W_emb0, b_emb0, W_emb1, b_emb1, W_t, b_t,
           W_s, b_s, W_h, b_h, W_o, b_o, W_out, b_out):
    n, d = x.shape
    e = edge_index.shape[1]
    nblk = n // _BN

    W_sh = jnp.concatenate([W_s, W_h], axis=0)
    b_sh = jnp.concatenate([b_s, b_h], axis=0)

    full = lambda shape: pl.BlockSpec(shape, lambda i: (0, 0))
    t_tab, sh_tab = pl.pallas_call(
        _tables_body,
        grid=(nblk,),
        in_specs=[
            pl.BlockSpec((_BN, d), lambda i: (i, 0)),
            full(W_emb0.shape), full((1, 128)),
            full(W_emb1.shape), full((1, 128)),
            full(W_t.shape), full((1, TD)),
            full(W_sh.shape), full((1, 2 * TD)),
        ],
        out_specs=[
            pl.BlockSpec((_BN, TD), lambda i: (i, 0)),
            pl.BlockSpec((_BN, 2 * TD), lambda i: (i, 0)),
        ],
        out_shape=[
            jax.ShapeDtypeStruct((n, TD), jnp.float32),
            jax.ShapeDtypeStruct((n, 2 * TD), jnp.float32),
        ],
    )(x, W_emb0, b_emb0.reshape(1, -1), W_emb1, b_emb1.reshape(1, -1),
      W_t, b_t.reshape(1, -1), W_sh, b_sh.reshape(1, -1))

    n_pad = ((n + _NS * 8 - 1) // (_NS * 8)) * (_NS * 8)
    mesh = plsc.VectorSubcoreMesh(core_axis_name="c", subcore_axis_name="s")
    sc_edge = functools.partial(
        pl.kernel,
        out_type=[
            jax.ShapeDtypeStruct((n_pad, ROWW), jnp.float32),
            jax.ShapeDtypeStruct((n_pad, ROWW), jnp.float32),
        ],
        mesh=mesh,
        compiler_params=pltpu.CompilerParams(use_tc_tiling_on_sc=False),
        scratch_types=[
            pltpu.VMEM((e // _C // (_NC * _NS), _C), jnp.int32),
            pltpu.VMEM((e // _C // (_NC * _NS), _C), jnp.int32),
            pltpu.VMEM((2, _C, TD), jnp.float32),
            pltpu.VMEM((2, _C, 2 * TD), jnp.float32),
            pltpu.VMEM((_C, ROWW), jnp.float32),
            pltpu.VMEM_SHARED((n_pad, ROWW), jnp.float32),
            pltpu.SemaphoreType.DMA,
            pltpu.SemaphoreType.DMA,
        ],
    )(_sc_edge_body)
    u0, u1 = sc_edge(edge_index[0].reshape(-1, _C),
                     edge_index[1].reshape(-1, _C), t_tab, sh_tab)

    logits = pl.pallas_call(
        _finish_body,
        grid=(nblk,),
        in_specs=[
            pl.BlockSpec((_BN, ROWW), lambda i: (i, 0)),
            pl.BlockSpec((_BN, ROWW), lambda i: (i, 0)),
            pl.BlockSpec((_BN, TD), lambda i: (i, 0)),
            pl.BlockSpec((_BN, 2 * TD), lambda i: (i, 0)),
            full(W_o.shape), full((1, W_o.shape[0])),
            full(W_out.shape), full((1, W_out.shape[0])),
        ],
        out_specs=pl.BlockSpec((_BN, W_out.shape[0]), lambda i: (i, 0)),
        out_shape=jax.ShapeDtypeStruct((n, W_out.shape[0]), jnp.float32),
    )(u0, u1, t_tab, sh_tab, W_o, b_o.reshape(1, -1),
      W_out, b_out.reshape(1, -1))
    return logits


# speedup vs baseline: 1.1891x; 1.0129x over previous
import functools

import jax
import jax.numpy as jnp
from jax import lax
from jax.experimental import pallas as pl
from jax.experimental.pallas import tpu as pltpu
from jax.experimental.pallas import tpu_sc as plsc

NV = 5
DV = 16
TD = NV * DV
ROWW = 96
_NC = 2
_NS = 16
_L = 16
_C = 80
_BN = 1000


def _tables_body(x_ref, w0_ref, b0_ref, w1_ref, b1_ref, wt_ref, bt_ref,
                 wsh_ref, bsh_ref, t_ref, sh_ref):
    dn = (((1,), (1,)), ((), ()))
    h = jnp.maximum(
        lax.dot_general(x_ref[...], w0_ref[...], dn,
                        preferred_element_type=jnp.float32) + b0_ref[...], 0.0)
    h = jnp.maximum(
        lax.dot_general(h, w1_ref[...], dn,
                        preferred_element_type=jnp.float32) + b1_ref[...], 0.0)
    t_ref[...] = jnp.maximum(
        lax.dot_general(h, wt_ref[...], dn,
                        preferred_element_type=jnp.float32) + bt_ref[...], 0.0)
    sh_ref[...] = jnp.maximum(
        lax.dot_general(h, wsh_ref[...], dn,
                        preferred_element_type=jnp.float32) + bsh_ref[...], 0.0)


def _sc_edge_body(src_hbm, dst_hbm, t_hbm, sh_hbm, out0, out1,
                  srcall, dstall, trow, shrow, mrow, spacc, sem_t, sem_s):
    cid = lax.axis_index("c")
    sid = lax.axis_index("s")
    wid = cid * _NS + sid
    n_pad = out0.shape[0]
    nrows = src_hbm.shape[0]
    kchunks = nrows // (_NC * _NS)
    rows_per_tile = n_pad // _NS
    lane = lax.iota(jnp.int32, _L)
    zero16 = jnp.zeros((_L,), jnp.float32)
    wbase = wid * kchunks

    def _zrow(i, carry):
        for j in range(ROWW // _L):
            mrow[i, pl.ds(_L * j, _L)] = zero16
        return carry
    lax.fori_loop(0, _C, _zrow, 0)
    base_r = sid * rows_per_tile
    nfull = rows_per_tile // _C
    rem = rows_per_tile - nfull * _C
    for k in range(nfull):
        pltpu.sync_copy(mrow, spacc.at[pl.ds(base_r + k * _C, _C)])
    if rem:
        pltpu.sync_copy(mrow.at[pl.ds(0, rem)],
                        spacc.at[pl.ds(base_r + nfull * _C, rem)])
    plsc.subcore_barrier()

    pltpu.sync_copy(src_hbm.at[pl.ds(wbase, kchunks)], srcall)
    pltpu.sync_copy(dst_hbm.at[pl.ds(wbase, kchunks)], dstall)

    dn = lax.GatherDimensionNumbers(
        offset_dims=(), collapsed_slice_dims=(0,), start_index_map=(0,))

    def _start(k, b):
        pltpu.async_copy(t_hbm.at[dstall.at[k]], trow.at[b], sem_t)
        pltpu.async_copy(sh_hbm.at[srcall.at[k]], shrow.at[b], sem_s)

    def _wait(k, b):
        pltpu.make_async_copy(t_hbm.at[dstall.at[k]], trow.at[b],
                              sem_t).wait()
        pltpu.make_async_copy(sh_hbm.at[srcall.at[k]], shrow.at[b],
                              sem_s).wait()

    def _compute_scatter(k, b):
        @plsc.parallel_loop(0, _C, 1, unroll=4)
        def _edge(i):
            pacc = zero16
            for v in range(NV):
                tv = trow[b, i, pl.ds(DV * v, DV)]
                sv = shrow[b, i, pl.ds(DV * v, DV)]
                r = tv * sv
                for step in (8, 4, 2, 1):
                    idx = (lane ^ step).reshape(_L, 1)
                    r = r + lax.gather(
                        r, idx, dn, (1,),
                        mode=lax.GatherScatterMode.PROMISE_IN_BOUNDS)
                pv = jnp.exp(r)
                hv = shrow[b, i, pl.ds(TD + DV * v, DV)]
                mrow[i, pl.ds(DV * v, DV)] = pv * hv
                pacc = jnp.where(lane == v, pv, pacc)
            mrow[i, pl.ds(TD, _L)] = pacc

        pltpu.sync_copy(mrow, spacc.at[dstall.at[k]], add=True)

    _start(0, 0)

    @pl.loop(0, kchunks - 1, step=2)
    def _pair(k):
        _start(k + 1, 1)
        _wait(k, 0)
        _compute_scatter(k, 0)
        _start(k + 2, 0)
        _wait(k + 1, 1)
        _compute_scatter(k + 1, 1)

    _wait(kchunks - 1, 0)
    _compute_scatter(kchunks - 1, 0)
    plsc.subcore_barrier()

    @pl.when(cid == 0)
    def _():
        pltpu.sync_copy(spacc.at[pl.ds(base_r, rows_per_tile)],
                        out0.at[pl.ds(base_r, rows_per_tile)])

    @pl.when(cid == 1)
    def _():
        pltpu.sync_copy(spacc.at[pl.ds(base_r, rows_per_tile)],
                        out1.at[pl.ds(base_r, rows_per_tile)])


def _finish_body(u0_ref, u1_ref, t_ref, sh_ref, wo_ref, bo_ref,
                 wout_ref, bout_ref, o_ref):
    acc = u0_ref[...] + u1_ref[...]
    tb = t_ref[...]
    shb = sh_ref[...]
    aggacc = jnp.zeros((o_ref.shape[0], DV), jnp.float32)
    for v in range(NV):
        prodv = tb[:, DV * v:DV * (v + 1)] * shb[:, DV * v:DV * (v + 1)]
        pv = jnp.exp(jnp.sum(prodv, axis=1, keepdims=True))
        hnv = shb[:, TD + DV * v:TD + DV * (v + 1)]
        uv = acc[:, DV * v:DV * (v + 1)] + pv * hnv
        normv = acc[:, TD + v:TD + v + 1] + pv + 1e-12
        aggacc = aggacc + uv / normv
    agg = aggacc * (1.0 / NV)
    dn = (((1,), (1,)), ((), ()))
    out = jnp.maximum(
        lax.dot_general(agg, wo_ref[...], dn,
                        preferred_element_type=jnp.float32) + bo_ref[...], 0.0)
    o_ref[...] = lax.dot_general(
        out, wout_ref[...], dn,
        preferred_element_type=jnp.float32) + bout_ref[...]


def kernel(x, edge_index, W_emb0, b_emb0, W_emb1, b_emb1, W_t, b_t,
           W_s, b_s, W_h, b_h, W_o, b_o, W_out, b_out):
    n, d = x.shape
    e = edge_index.shape[1]
    nblk = n // _BN

    W_sh = jnp.concatenate([W_s, W_h], axis=0)
    b_sh = jnp.concatenate([b_s, b_h], axis=0)

    full = lambda shape: pl.BlockSpec(shape, lambda i: (0, 0))
    t_tab, sh_tab = pl.pallas_call(
        _tables_body,
        grid=(nblk,),
        in_specs=[
            pl.BlockSpec((_BN, d), lambda i: (i, 0)),
            full(W_emb0.shape), full((1, 128)),
            full(W_emb1.shape), full((1, 128)),
            full(W_t.shape), full((1, TD)),
            full(W_sh.shape), full((1, 2 * TD)),
        ],
        out_specs=[
            pl.BlockSpec((_BN, TD), lambda i: (i, 0)),
            pl.BlockSpec((_BN, 2 * TD), lambda i: (i, 0)),
        ],
        out_shape=[
            jax.ShapeDtypeStruct((n, TD), jnp.float32),
            jax.ShapeDtypeStruct((n, 2 * TD), jnp.float32),
        ],
    )(x, W_emb0, b_emb0.reshape(1, -1), W_emb1, b_emb1.reshape(1, -1),
      W_t, b_t.reshape(1, -1), W_sh, b_sh.reshape(1, -1))

    n_pad = ((n + _NS * 8 - 1) // (_NS * 8)) * (_NS * 8)
    mesh = plsc.VectorSubcoreMesh(core_axis_name="c", subcore_axis_name="s")
    sc_edge = functools.partial(
        pl.kernel,
        out_type=[
            jax.ShapeDtypeStruct((n_pad, ROWW), jnp.float32),
            jax.ShapeDtypeStruct((n_pad, ROWW), jnp.float32),
        ],
        mesh=mesh,
        compiler_params=pltpu.CompilerParams(use_tc_tiling_on_sc=False),
        scratch_types=[
            pltpu.VMEM((e // _C // (_NC * _NS), _C), jnp.int32),
            pltpu.VMEM((e // _C // (_NC * _NS), _C), jnp.int32),
            pltpu.VMEM((2, _C, TD), jnp.float32),
            pltpu.VMEM((2, _C, 2 * TD), jnp.float32),
            pltpu.VMEM((_C, ROWW), jnp.float32),
            pltpu.VMEM_SHARED((n_pad, ROWW), jnp.float32),
            pltpu.SemaphoreType.DMA,
            pltpu.SemaphoreType.DMA,
        ],
    )(_sc_edge_body)
    u0, u1 = sc_edge(edge_index[0].reshape(-1, _C),
                     edge_index[1].reshape(-1, _C), t_tab, sh_tab)

    logits = pl.pallas_call(
        _finish_body,
        grid=(nblk,),
        in_specs=[
            pl.BlockSpec((_BN, ROWW), lambda i: (i, 0)),
            pl.BlockSpec((_BN, ROWW), lambda i: (i, 0)),
            pl.BlockSpec((_BN, TD), lambda i: (i, 0)),
            pl.BlockSpec((_BN, 2 * TD), lambda i: (i, 0)),
            full(W_o.shape), full((1, W_o.shape[0])),
            full(W_out.shape), full((1, W_out.shape[0])),
        ],
        out_specs=pl.BlockSpec((_BN, W_out.shape[0]), lambda i: (i, 0)),
        out_shape=jax.ShapeDtypeStruct((n, W_out.shape[0]), jnp.float32),
    )(u0, u1, t_tab, sh_tab, W_o, b_o.reshape(1, -1),
      W_out, b_out.reshape(1, -1))
    return logits


# speedup vs baseline: 1.2899x; 1.0847x over previous
import functools

import jax
import jax.numpy as jnp
from jax import lax
from jax.experimental import pallas as pl
from jax.experimental.pallas import tpu as pltpu
from jax.experimental.pallas import tpu_sc as plsc

NV = 5
DV = 16
TD = NV * DV
ROWW = 96
_NC = 2
_NS = 16
_L = 16
_C = 80
_BN = 2000


def _tables_body(x_ref, w0_ref, b0_ref, w1_ref, b1_ref, wt_ref, bt_ref,
                 wsh_ref, bsh_ref, t_ref, sh_ref):
    dn = (((1,), (1,)), ((), ()))
    h = jnp.maximum(
        lax.dot_general(x_ref[...], w0_ref[...], dn,
                        preferred_element_type=jnp.float32) + b0_ref[...], 0.0)
    h = jnp.maximum(
        lax.dot_general(h, w1_ref[...], dn,
                        preferred_element_type=jnp.float32) + b1_ref[...], 0.0)
    t_ref[...] = jnp.maximum(
        lax.dot_general(h, wt_ref[...], dn,
                        preferred_element_type=jnp.float32) + bt_ref[...], 0.0)
    sh_ref[...] = jnp.maximum(
        lax.dot_general(h, wsh_ref[...], dn,
                        preferred_element_type=jnp.float32) + bsh_ref[...], 0.0)


def _sc_edge_body(src_hbm, dst_hbm, t_hbm, sh_hbm, out0, out1,
                  srcall, dstall, trow, shrow, mrow, spacc, sem_t, sem_s):
    cid = lax.axis_index("c")
    sid = lax.axis_index("s")
    wid = cid * _NS + sid
    n_pad = out0.shape[0]
    nrows = src_hbm.shape[0]
    kchunks = nrows // (_NC * _NS)
    rows_per_tile = n_pad // _NS
    lane = lax.iota(jnp.int32, _L)
    zero16 = jnp.zeros((_L,), jnp.float32)
    wbase = wid * kchunks

    def _zrow(i, carry):
        for j in range(ROWW // _L):
            mrow[i, pl.ds(_L * j, _L)] = zero16
        return carry
    lax.fori_loop(0, _C, _zrow, 0)
    base_r = sid * rows_per_tile
    nfull = rows_per_tile // _C
    rem = rows_per_tile - nfull * _C
    for k in range(nfull):
        pltpu.sync_copy(mrow, spacc.at[pl.ds(base_r + k * _C, _C)])
    if rem:
        pltpu.sync_copy(mrow.at[pl.ds(0, rem)],
                        spacc.at[pl.ds(base_r + nfull * _C, rem)])
    plsc.subcore_barrier()

    pltpu.sync_copy(src_hbm.at[pl.ds(wbase, kchunks)], srcall)
    pltpu.sync_copy(dst_hbm.at[pl.ds(wbase, kchunks)], dstall)

    dn = lax.GatherDimensionNumbers(
        offset_dims=(), collapsed_slice_dims=(0,), start_index_map=(0,))

    def _start(k, b):
        pltpu.async_copy(t_hbm.at[dstall.at[k]], trow.at[b], sem_t)
        pltpu.async_copy(sh_hbm.at[srcall.at[k]], shrow.at[b], sem_s)

    def _wait(k, b):
        pltpu.make_async_copy(t_hbm.at[dstall.at[k]], trow.at[b],
                              sem_t).wait()
        pltpu.make_async_copy(sh_hbm.at[srcall.at[k]], shrow.at[b],
                              sem_s).wait()

    def _compute_scatter(k, b):
        @plsc.parallel_loop(0, _C, 1, unroll=4)
        def _edge(i):
            pacc = zero16
            for v in range(NV):
                tv = trow[b, i, pl.ds(DV * v, DV)]
                sv = shrow[b, i, pl.ds(DV * v, DV)]
                r = tv * sv
                for step in (8, 4, 2, 1):
                    idx = (lane ^ step).reshape(_L, 1)
                    r = r + lax.gather(
                        r, idx, dn, (1,),
                        mode=lax.GatherScatterMode.PROMISE_IN_BOUNDS)
                pv = jnp.exp(r)
                hv = shrow[b, i, pl.ds(TD + DV * v, DV)]
                mrow[i, pl.ds(DV * v, DV)] = pv * hv
                pacc = jnp.where(lane == v, pv, pacc)
            mrow[i, pl.ds(TD, _L)] = pacc

        pltpu.sync_copy(mrow, spacc.at[dstall.at[k]], add=True)

    _start(0, 0)

    @pl.loop(0, kchunks - 1, step=2)
    def _pair(k):
        _start(k + 1, 1)
        _wait(k, 0)
        _compute_scatter(k, 0)
        _start(k + 2, 0)
        _wait(k + 1, 1)
        _compute_scatter(k + 1, 1)

    _wait(kchunks - 1, 0)
    _compute_scatter(kchunks - 1, 0)
    plsc.subcore_barrier()

    @pl.when(cid == 0)
    def _():
        pltpu.sync_copy(spacc.at[pl.ds(base_r, rows_per_tile)],
                        out0.at[pl.ds(base_r, rows_per_tile)])

    @pl.when(cid == 1)
    def _():
        pltpu.sync_copy(spacc.at[pl.ds(base_r, rows_per_tile)],
                        out1.at[pl.ds(base_r, rows_per_tile)])


def _finish_body(u0_ref, u1_ref, t_ref, sh_ref, wo_ref, bo_ref,
                 wout_ref, bout_ref, o_ref):
    acc = u0_ref[...] + u1_ref[...]
    tb = t_ref[...]
    shb = sh_ref[...]
    aggacc = jnp.zeros((o_ref.shape[0], DV), jnp.float32)
    for v in range(NV):
        prodv = tb[:, DV * v:DV * (v + 1)] * shb[:, DV * v:DV * (v + 1)]
        pv = jnp.exp(jnp.sum(prodv, axis=1, keepdims=True))
        hnv = shb[:, TD + DV * v:TD + DV * (v + 1)]
        uv = acc[:, DV * v:DV * (v + 1)] + pv * hnv
        normv = acc[:, TD + v:TD + v + 1] + pv + 1e-12
        aggacc = aggacc + uv / normv
    agg = aggacc * (1.0 / NV)
    dn = (((1,), (1,)), ((), ()))
    out = jnp.maximum(
        lax.dot_general(agg, wo_ref[...], dn,
                        preferred_element_type=jnp.float32) + bo_ref[...], 0.0)
    o_ref[...] = lax.dot_general(
        out, wout_ref[...], dn,
        preferred_element_type=jnp.float32) + bout_ref[...]


def kernel(x, edge_index, W_emb0, b_emb0, W_emb1, b_emb1, W_t, b_t,
           W_s, b_s, W_h, b_h, W_o, b_o, W_out, b_out):
    n, d = x.shape
    e = edge_index.shape[1]
    nblk = n // _BN

    W_sh = jnp.concatenate([W_s, W_h], axis=0)
    b_sh = jnp.concatenate([b_s, b_h], axis=0)

    full = lambda shape: pl.BlockSpec(shape, lambda i: (0, 0))
    t_tab, sh_tab = pl.pallas_call(
        _tables_body,
        grid=(nblk,),
        in_specs=[
            pl.BlockSpec((_BN, d), lambda i: (i, 0)),
            full(W_emb0.shape), full((1, 128)),
            full(W_emb1.shape), full((1, 128)),
            full(W_t.shape), full((1, TD)),
            full(W_sh.shape), full((1, 2 * TD)),
        ],
        out_specs=[
            pl.BlockSpec((_BN, TD), lambda i: (i, 0)),
            pl.BlockSpec((_BN, 2 * TD), lambda i: (i, 0)),
        ],
        out_shape=[
            jax.ShapeDtypeStruct((n, TD), jnp.float32),
            jax.ShapeDtypeStruct((n, 2 * TD), jnp.float32),
        ],
    )(x, W_emb0, b_emb0.reshape(1, -1), W_emb1, b_emb1.reshape(1, -1),
      W_t, b_t.reshape(1, -1), W_sh, b_sh.reshape(1, -1))

    n_pad = ((n + _NS * 8 - 1) // (_NS * 8)) * (_NS * 8)
    mesh = plsc.VectorSubcoreMesh(core_axis_name="c", subcore_axis_name="s")
    sc_edge = functools.partial(
        pl.kernel,
        out_type=[
            jax.ShapeDtypeStruct((n_pad, ROWW), jnp.float32),
            jax.ShapeDtypeStruct((n_pad, ROWW), jnp.float32),
        ],
        mesh=mesh,
        compiler_params=pltpu.CompilerParams(use_tc_tiling_on_sc=False),
        scratch_types=[
            pltpu.VMEM((e // _C // (_NC * _NS), _C), jnp.int32),
            pltpu.VMEM((e // _C // (_NC * _NS), _C), jnp.int32),
            pltpu.VMEM((2, _C, TD), jnp.float32),
            pltpu.VMEM((2, _C, 2 * TD), jnp.float32),
            pltpu.VMEM((_C, ROWW), jnp.float32),
            pltpu.VMEM_SHARED((n_pad, ROWW), jnp.float32),
            pltpu.SemaphoreType.DMA,
            pltpu.SemaphoreType.DMA,
        ],
    )(_sc_edge_body)
    u0, u1 = sc_edge(edge_index[0].reshape(-1, _C),
                     edge_index[1].reshape(-1, _C), t_tab, sh_tab)

    logits = pl.pallas_call(
        _finish_body,
        grid=(nblk,),
        in_specs=[
            pl.BlockSpec((_BN, ROWW), lambda i: (i, 0)),
            pl.BlockSpec((_BN, ROWW), lambda i: (i, 0)),
            pl.BlockSpec((_BN, TD), lambda i: (i, 0)),
            pl.BlockSpec((_BN, 2 * TD), lambda i: (i, 0)),
            full(W_o.shape), full((1, W_o.shape[0])),
            full(W_out.shape), full((1, W_out.shape[0])),
        ],
        out_specs=pl.BlockSpec((_BN, W_out.shape[0]), lambda i: (i, 0)),
        out_shape=jax.ShapeDtypeStruct((n, W_out.shape[0]), jnp.float32),
    )(u0, u1, t_tab, sh_tab, W_o, b_o.reshape(1, -1),
      W_out, b_out.reshape(1, -1))
    return logits
